# baseline (speedup 1.0000x reference)
import functools

import jax
import jax.numpy as jnp
from jax import lax
from jax.experimental import pallas as pl
from jax.experimental.pallas import tpu as pltpu
from jax.experimental.pallas import tpu_sc as plsc

_D = 128
_NW = 32
_CHUNK = 80
_NBUF = 10
_NPAIR = _NBUF // 2
_NGRP = 4
_NCHUNK = _NBUF * _NGRP
_PER_W = _CHUNK * _NCHUNK
_B = 100000
_B_PAD = _PER_W * _NW
_LAST_FULL = (_B - 31 * _PER_W) // _CHUNK
_LAST_REM = _B - 31 * _PER_W - _LAST_FULL * _CHUNK

_mesh = plsc.VectorSubcoreMesh(core_axis_name="c", subcore_axis_name="s")


@functools.partial(
    pl.kernel,
    out_type=jax.ShapeDtypeStruct((_B, _D), jnp.float32),
    mesh=_mesh,
    scratch_types=[
        pltpu.VMEM((_NCHUNK, _CHUNK), jnp.int32),
        pltpu.VMEM((_NBUF * _CHUNK, _D), jnp.float32),
        [pltpu.SemaphoreType.DMA for _ in range(_NBUF)],
        [pltpu.SemaphoreType.DMA for _ in range(_NPAIR)],
    ],
)
def _sc_gather(x_hbm, idx_hbm, out_hbm, idx_v, big, gsems, ssems):
    wid = lax.axis_index("s") * 2 + lax.axis_index("c")
    base = wid * _PER_W
    pltpu.sync_copy(idx_hbm.at[wid], idx_v)

    def fire_gather(c, b):
        pltpu.async_copy(
            x_hbm.at[idx_v.at[c]], big.at[pl.ds(b * _CHUNK, _CHUNK)], gsems[b]
        )

    def fire_scatter_pair(p, q):
        pltpu.async_copy(
            big.at[pl.ds(2 * q * _CHUNK, 2 * _CHUNK)],
            out_hbm.at[pl.ds(base + 2 * p * _CHUNK, 2 * _CHUNK)],
            ssems[q],
        )

    def drain(sem, rows):
        pltpu.make_async_copy(
            x_hbm.at[pl.ds(0, rows)], big.at[pl.ds(0, rows)], sem
        ).wait()

    @pl.when(wid < _NW - 1)
    def _full_worker():
        for b in range(_NBUF):
            fire_gather(b, b)

        def body(i, carry):
            for q in range(_NPAIR):
                drain(gsems[2 * q], _CHUNK)
                drain(gsems[2 * q + 1], _CHUNK)
                fire_scatter_pair(i * _NPAIR + q, q)
                if q >= 2:
                    qq = q - 2
                    drain(ssems[qq], 2 * _CHUNK)
                    fire_gather((i + 1) * _NBUF + 2 * qq, 2 * qq)
                    fire_gather((i + 1) * _NBUF + 2 * qq + 1, 2 * qq + 1)
            for qq in range(_NPAIR - 2, _NPAIR):
                drain(ssems[qq], 2 * _CHUNK)
                fire_gather((i + 1) * _NBUF + 2 * qq, 2 * qq)
                fire_gather((i + 1) * _NBUF + 2 * qq + 1, 2 * qq + 1)
            return carry

        lax.fori_loop(0, _NGRP - 1, body, 0)

        for q in range(_NPAIR):
            drain(gsems[2 * q], _CHUNK)
            drain(gsems[2 * q + 1], _CHUNK)
            fire_scatter_pair((_NGRP - 1) * _NPAIR + q, q)
        for q in range(_NPAIR):
            drain(ssems[q], 2 * _CHUNK)

    @pl.when(wid == _NW - 1)
    def _tail_worker():
        def body(c, carry):
            pltpu.async_copy(
                x_hbm.at[idx_v.at[c]], big.at[pl.ds(0, _CHUNK)], gsems[0]
            ).wait()
            pltpu.sync_copy(
                big.at[pl.ds(0, _CHUNK)],
                out_hbm.at[pl.ds(base + c * _CHUNK, _CHUNK)],
            )
            return carry

        lax.fori_loop(0, _LAST_FULL, body, 0)
        if _LAST_REM:
            pltpu.async_copy(
                x_hbm.at[idx_v.at[_LAST_FULL]], big.at[pl.ds(0, _CHUNK)], gsems[0]
            ).wait()
            pltpu.sync_copy(
                big.at[pl.ds(0, _LAST_REM)],
                out_hbm.at[pl.ds(base + _LAST_FULL * _CHUNK, _LAST_REM)],
            )


def kernel(x, upsamples):
    n = upsamples.shape[0]
    idx = upsamples[:, 0].astype(jnp.int32)
    idx = jnp.concatenate([idx, jnp.zeros((_B_PAD - n,), jnp.int32)])
    idx = idx.reshape(_NW, _NCHUNK, _CHUNK)
    out = _sc_gather(x, idx)
    return out


# speedup vs baseline: 1.0127x; 1.0127x over previous
import functools

import jax
import jax.numpy as jnp
from jax import lax
from jax.experimental import pallas as pl
from jax.experimental.pallas import tpu as pltpu
from jax.experimental.pallas import tpu_sc as plsc

_D = 128
_NW = 32
_CHUNK = 80
_NBUF = 10
_NPAIR = _NBUF // 2
_NGRP = 4
_NCHUNK = _NBUF * _NGRP
_PER_W = _CHUNK * _NCHUNK
_B = 100000
_B_PAD = _PER_W * _NW
_LAST_FULL = (_B - 31 * _PER_W) // _CHUNK
_LAST_REM = _B - 31 * _PER_W - _LAST_FULL * _CHUNK

_mesh = plsc.VectorSubcoreMesh(core_axis_name="c", subcore_axis_name="s")


@functools.partial(
    pl.kernel,
    out_type=jax.ShapeDtypeStruct((_B, _D), jnp.float32),
    mesh=_mesh,
    scratch_types=[
        pltpu.VMEM((_PER_W,), jnp.int32),
        pltpu.VMEM((_NBUF * _CHUNK, _D), jnp.float32),
        [pltpu.SemaphoreType.DMA for _ in range(_NBUF)],
        [pltpu.SemaphoreType.DMA for _ in range(_NPAIR)],
    ],
)
def _sc_gather(x_hbm, idx_hbm, out_hbm, idx_v, big, gsems, ssems):
    wid = lax.axis_index("s") * 2 + lax.axis_index("c")
    base = wid * _PER_W

    def fire_gather(c, b):
        pltpu.async_copy(
            x_hbm.at[idx_v.at[pl.ds(c * _CHUNK, _CHUNK)]],
            big.at[pl.ds(b * _CHUNK, _CHUNK)],
            gsems[b],
        )

    def fire_scatter_pair(p, q):
        pltpu.async_copy(
            big.at[pl.ds(2 * q * _CHUNK, 2 * _CHUNK)],
            out_hbm.at[pl.ds(base + 2 * p * _CHUNK, 2 * _CHUNK)],
            ssems[q],
        )

    def drain(sem, rows):
        pltpu.make_async_copy(
            x_hbm.at[pl.ds(0, rows)], big.at[pl.ds(0, rows)], sem
        ).wait()

    @pl.when(wid < _NW - 1)
    def _full_worker():
        pltpu.sync_copy(idx_hbm.at[pl.ds(base, _PER_W)], idx_v)
        for b in range(_NBUF):
            fire_gather(b, b)

        def body(i, carry):
            for q in range(_NPAIR):
                drain(gsems[2 * q], _CHUNK)
                drain(gsems[2 * q + 1], _CHUNK)
                fire_scatter_pair(i * _NPAIR + q, q)
                if q >= 2:
                    qq = q - 2
                    drain(ssems[qq], 2 * _CHUNK)
                    fire_gather((i + 1) * _NBUF + 2 * qq, 2 * qq)
                    fire_gather((i + 1) * _NBUF + 2 * qq + 1, 2 * qq + 1)
            for qq in range(_NPAIR - 2, _NPAIR):
                drain(ssems[qq], 2 * _CHUNK)
                fire_gather((i + 1) * _NBUF + 2 * qq, 2 * qq)
                fire_gather((i + 1) * _NBUF + 2 * qq + 1, 2 * qq + 1)
            return carry

        lax.fori_loop(0, _NGRP - 1, body, 0)

        for q in range(_NPAIR):
            drain(gsems[2 * q], _CHUNK)
            drain(gsems[2 * q + 1], _CHUNK)
            fire_scatter_pair((_NGRP - 1) * _NPAIR + q, q)
        for q in range(_NPAIR):
            drain(ssems[q], 2 * _CHUNK)

    _TAIL = _B - 31 * _PER_W

    @pl.when(wid == _NW - 1)
    def _tail_worker():
        pltpu.sync_copy(idx_hbm.at[pl.ds(base, _TAIL)], idx_v.at[pl.ds(0, _TAIL)])

        def body(c, carry):
            pltpu.async_copy(
                x_hbm.at[idx_v.at[pl.ds(c * _CHUNK, _CHUNK)]],
                big.at[pl.ds(0, _CHUNK)],
                gsems[0],
            ).wait()
            pltpu.sync_copy(
                big.at[pl.ds(0, _CHUNK)],
                out_hbm.at[pl.ds(base + c * _CHUNK, _CHUNK)],
            )
            return carry

        lax.fori_loop(0, _LAST_FULL, body, 0)
        if _LAST_REM:
            pltpu.async_copy(
                x_hbm.at[idx_v.at[pl.ds(_LAST_FULL * _CHUNK, _LAST_REM)]],
                big.at[pl.ds(0, _LAST_REM)],
                gsems[0],
            ).wait()
            pltpu.sync_copy(
                big.at[pl.ds(0, _LAST_REM)],
                out_hbm.at[pl.ds(base + _LAST_FULL * _CHUNK, _LAST_REM)],
            )


def kernel(x, upsamples):
    idx = upsamples[:, 0].astype(jnp.int32)
    out = _sc_gather(x, idx)
    return out
